# baseline (speedup 1.0000x reference)
import functools

import jax
import jax.numpy as jnp
from jax import lax
from jax.experimental import pallas as pl
from jax.experimental.pallas import tpu as pltpu
from jax.experimental.pallas import tpu_sc as plsc

D_MODEL = 1024
N_EXPERTS = 8
RANK = 256
TOK = 2048
T3 = 256
NPAD = 2 * TOK + N_EXPERTS * T3
NTILES = NPAD // T3
NW = 32
CH = TOK // NW
SUB = 32


def _router_kernel(x_ref, wg_ref, pos0_ref, pos1_ref, w1r_ref, w2r_ref, te_ref):
    xf = x_ref[...]
    logits = jnp.dot(xf, wg_ref[...].T, preferred_element_type=jnp.float32)
    idx = lax.broadcasted_iota(jnp.int32, logits.shape, 1)
    m1 = jnp.max(logits, axis=-1, keepdims=True)
    a1 = jnp.min(jnp.where(logits == m1, idx, N_EXPERTS), axis=-1, keepdims=True)
    logits2 = jnp.where(idx == a1, -jnp.inf, logits)
    m2 = jnp.max(logits2, axis=-1, keepdims=True)
    a2 = jnp.min(jnp.where(logits2 == m2, idx, N_EXPERTS), axis=-1, keepdims=True)
    t = jnp.exp(m2 - m1)
    w1 = 1.0 / (1.0 + t)
    w2 = t / (1.0 + t)

    one0 = (idx == a1).astype(jnp.float32)
    one1 = (idx == a2).astype(jnp.float32)

    c = 256
    r_ = lax.broadcasted_iota(jnp.int32, (c, c), 0)
    c_ = lax.broadcasted_iota(jnp.int32, (c, c), 1)
    ltri = (c_ < r_).astype(jnp.float32)
    rank0_chunks = []
    rank1_chunks = []
    run0 = jnp.zeros((1, N_EXPERTS), jnp.float32)
    run1 = jnp.zeros((1, N_EXPERTS), jnp.float32)
    for k in range(TOK // c):
        o0 = one0[k * c:(k + 1) * c]
        o1 = one1[k * c:(k + 1) * c]
        rank0_chunks.append(
            jnp.dot(ltri, o0, preferred_element_type=jnp.float32) + run0)
        rank1_chunks.append(
            jnp.dot(ltri, o1, preferred_element_type=jnp.float32) + run1)
        run0 = run0 + jnp.sum(o0, axis=0, keepdims=True)
        run1 = run1 + jnp.sum(o1, axis=0, keepdims=True)
    rank0 = jnp.concatenate(rank0_chunks, axis=0)
    rank1 = jnp.concatenate(rank1_chunks, axis=0)

    counts = run0 + run1
    pc = jnp.ceil(counts / T3) * T3
    e_ = lax.broadcasted_iota(jnp.int32, (N_EXPERTS, N_EXPERTS), 0)
    f_ = lax.broadcasted_iota(jnp.int32, (N_EXPERTS, N_EXPERTS), 1)
    sutri = (e_ < f_).astype(jnp.float32)
    starts = jnp.dot(pc, sutri, preferred_element_type=jnp.float32)

    pos0 = jnp.sum(one0 * (starts + rank0), axis=-1, keepdims=True)
    pos1 = jnp.sum(one1 * (starts + run0 + rank1), axis=-1, keepdims=True)
    pos0_ref[...] = pos0.astype(jnp.int32)
    pos1_ref[...] = pos1.astype(jnp.int32)
    w1r_ref[...] = jnp.broadcast_to(w1, (TOK, 128))
    w2r_ref[...] = jnp.broadcast_to(w2, (TOK, 128))

    bnd = starts + pc
    ti = lax.broadcasted_iota(jnp.int32, (NTILES, N_EXPERTS), 0) * T3
    te = jnp.sum((ti >= bnd).astype(jnp.int32), axis=-1, keepdims=True)
    te_ref[...] = jnp.minimum(te, N_EXPERTS - 1)


def _run_router(x2, Wg):
    return pl.pallas_call(
        _router_kernel,
        grid=(1,),
        in_specs=[
            pl.BlockSpec((TOK, D_MODEL), lambda i: (0, 0)),
            pl.BlockSpec((N_EXPERTS, D_MODEL), lambda i: (0, 0)),
        ],
        out_specs=[
            pl.BlockSpec((TOK, 1), lambda i: (0, 0)),
            pl.BlockSpec((TOK, 1), lambda i: (0, 0)),
            pl.BlockSpec((TOK, 128), lambda i: (0, 0)),
            pl.BlockSpec((TOK, 128), lambda i: (0, 0)),
            pl.BlockSpec((NTILES, 1), lambda i: (0, 0)),
        ],
        out_shape=[
            jax.ShapeDtypeStruct((TOK, 1), jnp.int32),
            jax.ShapeDtypeStruct((TOK, 1), jnp.int32),
            jax.ShapeDtypeStruct((TOK, 128), jnp.float32),
            jax.ShapeDtypeStruct((TOK, 128), jnp.float32),
            jax.ShapeDtypeStruct((NTILES, 1), jnp.int32),
        ],
    )(x2, Wg)


def _make_dispatch():
    mesh = plsc.VectorSubcoreMesh(core_axis_name="c", subcore_axis_name="s")

    @functools.partial(
        pl.kernel, mesh=mesh,
        out_type=[
            jax.ShapeDtypeStruct((NPAD, D_MODEL // 2), jnp.int32),
            jax.ShapeDtypeStruct((NPAD, 128), jnp.float32),
        ],
        scratch_types=[
            pltpu.VMEM((CH,), jnp.int32),
            pltpu.VMEM((CH,), jnp.int32),
            pltpu.VMEM((CH, D_MODEL // 2), jnp.int32),
            pltpu.VMEM((CH, 128), jnp.float32),
            pltpu.VMEM((CH, 128), jnp.float32),
            pltpu.SemaphoreType.DMA,
            pltpu.SemaphoreType.DMA,
            pltpu.SemaphoreType.DMA,
            pltpu.SemaphoreType.DMA,
        ],
    )
    def dispatch(xb_hbm, pos0_hbm, pos1_hbm, w1r_hbm, w2r_hbm, xs_hbm, ws_hbm,
                 idx0_v, idx1_v, rows_v, wr1_v, wr2_v,
                 sem0, sem1, sem2, sem3):
        wid = lax.axis_index("s") * 2 + lax.axis_index("c")
        base = wid * CH
        pltpu.sync_copy(pos0_hbm.at[pl.ds(base, CH)], idx0_v)
        pltpu.sync_copy(pos1_hbm.at[pl.ds(base, CH)], idx1_v)
        pltpu.sync_copy(w1r_hbm.at[pl.ds(base, CH)], wr1_v)
        pltpu.sync_copy(w2r_hbm.at[pl.ds(base, CH)], wr2_v)
        pltpu.sync_copy(xb_hbm.at[pl.ds(base, CH)], rows_v)

        c0 = pltpu.async_copy(rows_v, xs_hbm.at[idx0_v], sem0)
        c1 = pltpu.async_copy(rows_v, xs_hbm.at[idx1_v], sem1)
        c2 = pltpu.async_copy(wr1_v, ws_hbm.at[idx0_v], sem2)
        c3 = pltpu.async_copy(wr2_v, ws_hbm.at[idx1_v], sem3)
        c0.wait()
        c1.wait()
        c2.wait()
        c3.wait()

    return dispatch


def _expert_kernel(te_ref, xs_ref, ws_ref, wu_ref, wv_ref, wo_ref, ys_ref):
    xt = xs_ref[...]
    u = jnp.dot(xt, wu_ref[0].T, preferred_element_type=jnp.float32)
    v = jnp.dot(xt, wv_ref[0].T, preferred_element_type=jnp.float32)
    s = u * jax.nn.sigmoid(u) * v
    s = s * ws_ref[...][:, 0:1]
    ys_ref[...] = jnp.dot(s.astype(jnp.bfloat16), wo_ref[0],
                          preferred_element_type=jnp.float32)


def _run_experts(xs, ws, wu, wv, wo_t, te):
    grid_spec = pltpu.PrefetchScalarGridSpec(
        num_scalar_prefetch=1,
        grid=(NTILES,),
        in_specs=[
            pl.BlockSpec((T3, D_MODEL), lambda i, te: (i, 0)),
            pl.BlockSpec((T3, 128), lambda i, te: (i, 0)),
            pl.BlockSpec((1, RANK, D_MODEL), lambda i, te: (te[i], 0, 0)),
            pl.BlockSpec((1, RANK, D_MODEL), lambda i, te: (te[i], 0, 0)),
            pl.BlockSpec((1, RANK, D_MODEL), lambda i, te: (te[i], 0, 0)),
        ],
        out_specs=pl.BlockSpec((T3, D_MODEL), lambda i, te: (i, 0)),
    )
    return pl.pallas_call(
        _expert_kernel,
        grid_spec=grid_spec,
        out_shape=jax.ShapeDtypeStruct((NPAD, D_MODEL), jnp.float32),
    )(te, xs, ws, wu, wv, wo_t)


def _make_combine():
    mesh = plsc.VectorSubcoreMesh(core_axis_name="c", subcore_axis_name="s")

    @functools.partial(
        pl.kernel, mesh=mesh,
        out_type=jax.ShapeDtypeStruct((TOK, D_MODEL), jnp.float32),
        scratch_types=[
            pltpu.VMEM((SUB,), jnp.int32),
            pltpu.VMEM((SUB,), jnp.int32),
            pltpu.VMEM((SUB, D_MODEL), jnp.float32),
            pltpu.VMEM((SUB, D_MODEL), jnp.float32),
            pltpu.SemaphoreType.DMA,
            pltpu.SemaphoreType.DMA,
        ],
    )
    def combine(ys_hbm, pos0_hbm, pos1_hbm, out_hbm, idx0_v, idx1_v, buf0_v,
                buf1_v, sem0, sem1):
        wid = lax.axis_index("s") * 2 + lax.axis_index("c")
        base = wid * CH
        nvec = D_MODEL // 16
        for sub in range(CH // SUB):
            off = base + sub * SUB
            pltpu.sync_copy(pos0_hbm.at[pl.ds(off, SUB)], idx0_v)
            pltpu.sync_copy(pos1_hbm.at[pl.ds(off, SUB)], idx1_v)
            c0 = pltpu.async_copy(ys_hbm.at[idx0_v], buf0_v, sem0)
            c1 = pltpu.async_copy(ys_hbm.at[idx1_v], buf1_v, sem1)
            c0.wait()
            c1.wait()

            def body(t, _):
                for i in range(nvec):
                    sl = pl.ds(i * 16, 16)
                    buf0_v[t, sl] = buf0_v[t, sl] + buf1_v[t, sl]
                return 0
            lax.fori_loop(0, SUB, body, 0)
            pltpu.sync_copy(buf0_v, out_hbm.at[pl.ds(off, SUB)])

    return combine


def kernel(x, Wg, Wu, Wv, Wo):
    B, N, D = x.shape
    x2 = x.reshape(B * N, D)
    xb = x2.astype(jnp.bfloat16)

    pos0, pos1, w1r, w2r, te = _run_router(x2, Wg)
    pos0 = pos0.reshape(TOK)
    pos1 = pos1.reshape(TOK)
    te = te.reshape(NTILES)

    xbi = lax.bitcast_convert_type(xb.reshape(TOK, D // 2, 2), jnp.int32)
    xs_i, ws = _make_dispatch()(xbi, pos0, pos1, w1r, w2r)
    xs = lax.bitcast_convert_type(xs_i, jnp.bfloat16).reshape(NPAD, D)

    wu = Wu.astype(jnp.bfloat16)
    wv = Wv.astype(jnp.bfloat16)
    wo_t = jnp.transpose(Wo, (0, 2, 1)).astype(jnp.bfloat16)
    ys = _run_experts(xs, ws, wu, wv, wo_t, te)

    out = _make_combine()(ys, pos0, pos1)
    return out.reshape(B, N, D)


# speedup vs baseline: 4.4445x; 4.4445x over previous
import jax
import jax.numpy as jnp
from jax.experimental import pallas as pl

D_MODEL = 1024
N_EXPERTS = 8
RANK = 256
TOKEN_TILE = 512


def _moe_dense_kernel(x_ref, xb_ref, wg_ref, wuv_ref, wo_ref, out_ref):
    xf = x_ref[...]
    xb = xb_ref[...]

    logits = jnp.dot(xf, wg_ref[...].T, preferred_element_type=jnp.float32)
    idx = jax.lax.broadcasted_iota(jnp.int32, logits.shape, 1)
    m1 = jnp.max(logits, axis=-1, keepdims=True)
    a1 = jnp.min(jnp.where(logits == m1, idx, N_EXPERTS), axis=-1, keepdims=True)
    logits2 = jnp.where(idx == a1, -jnp.inf, logits)
    m2 = jnp.max(logits2, axis=-1, keepdims=True)
    a2 = jnp.min(jnp.where(logits2 == m2, idx, N_EXPERTS), axis=-1, keepdims=True)
    t = jnp.exp(m2 - m1)
    w1 = 1.0 / (1.0 + t)
    w2 = t / (1.0 + t)

    uv = jnp.dot(xb, wuv_ref[...].T, preferred_element_type=jnp.float32)
    u = uv[:, :N_EXPERTS * RANK]
    v = uv[:, N_EXPERTS * RANK:]
    s = u * jax.nn.sigmoid(u) * v

    blocks = []
    for e in range(N_EXPERTS):
        ce = w1 * (a1 == e) + w2 * (a2 == e)
        blocks.append((ce * s[:, e * RANK:(e + 1) * RANK]).astype(jnp.bfloat16))
    s_all = jnp.concatenate(blocks, axis=1)

    out_ref[...] = jnp.dot(s_all, wo_ref[...], preferred_element_type=jnp.float32)


def kernel(x, Wg, Wu, Wv, Wo):
    B, N, D = x.shape
    x2 = x.reshape(B * N, D)
    xb = x2.astype(jnp.bfloat16)
    nt = (B * N) // TOKEN_TILE
    ER = N_EXPERTS * RANK

    wuv_all = jnp.concatenate([Wu.reshape(ER, D), Wv.reshape(ER, D)],
                              axis=0).astype(jnp.bfloat16)
    wo_all = jnp.transpose(Wo, (0, 2, 1)).reshape(ER, D).astype(jnp.bfloat16)

    out = pl.pallas_call(
        _moe_dense_kernel,
        grid=(nt,),
        in_specs=[
            pl.BlockSpec((TOKEN_TILE, D), lambda i: (i, 0)),
            pl.BlockSpec((TOKEN_TILE, D), lambda i: (i, 0)),
            pl.BlockSpec((N_EXPERTS, D), lambda i: (0, 0)),
            pl.BlockSpec((2 * ER, D), lambda i: (0, 0)),
            pl.BlockSpec((ER, D), lambda i: (0, 0)),
        ],
        out_specs=pl.BlockSpec((TOKEN_TILE, D), lambda i: (i, 0)),
        out_shape=jax.ShapeDtypeStruct((B * N, D), x.dtype),
    )(x2, xb, Wg, wuv_all, wo_all)
    return out.reshape(B, N, D)


# speedup vs baseline: 5.0137x; 1.1281x over previous
import jax
import jax.numpy as jnp
from jax.experimental import pallas as pl

D_MODEL = 1024
N_EXPERTS = 8
RANK = 256
TOKEN_TILE = 512


def _moe_dense_kernel(x_ref, xb_ref, wg_ref, wu_ref, wv_ref, wo_ref, out_ref):
    xf = x_ref[...]
    xb = xb_ref[...]

    logits = jnp.dot(xf, wg_ref[...].T, preferred_element_type=jnp.float32)
    idx = jax.lax.broadcasted_iota(jnp.int32, logits.shape, 1)
    m1 = jnp.max(logits, axis=-1, keepdims=True)
    a1 = jnp.min(jnp.where(logits == m1, idx, N_EXPERTS), axis=-1, keepdims=True)
    logits2 = jnp.where(idx == a1, -jnp.inf, logits)
    m2 = jnp.max(logits2, axis=-1, keepdims=True)
    a2 = jnp.min(jnp.where(logits2 == m2, idx, N_EXPERTS), axis=-1, keepdims=True)
    t = jnp.exp(m2 - m1)
    w1 = 1.0 / (1.0 + t)
    w2 = t / (1.0 + t)

    u = jnp.dot(xb, wu_ref[...].T, preferred_element_type=jnp.float32)
    v = jnp.dot(xb, wv_ref[...].T, preferred_element_type=jnp.float32)
    s = u * jax.nn.sigmoid(u) * v

    blocks = []
    for e in range(N_EXPERTS):
        ce = w1 * (a1 == e) + w2 * (a2 == e)
        blocks.append((ce * s[:, e * RANK:(e + 1) * RANK]).astype(jnp.bfloat16))
    s_all = jnp.concatenate(blocks, axis=1)

    out_ref[...] = jnp.dot(s_all, wo_ref[...], preferred_element_type=jnp.float32)


def kernel(x, Wg, Wu, Wv, Wo):
    B, N, D = x.shape
    x2 = x.reshape(B * N, D)
    xb = x2.astype(jnp.bfloat16)
    nt = (B * N) // TOKEN_TILE
    ER = N_EXPERTS * RANK

    wu_all = Wu.reshape(ER, D).astype(jnp.bfloat16)
    wv_all = Wv.reshape(ER, D).astype(jnp.bfloat16)
    wo_all = jnp.transpose(Wo, (0, 2, 1)).reshape(ER, D).astype(jnp.bfloat16)

    out = pl.pallas_call(
        _moe_dense_kernel,
        grid=(nt,),
        in_specs=[
            pl.BlockSpec((TOKEN_TILE, D), lambda i: (i, 0)),
            pl.BlockSpec((TOKEN_TILE, D), lambda i: (i, 0)),
            pl.BlockSpec((N_EXPERTS, D), lambda i: (0, 0)),
            pl.BlockSpec((ER, D), lambda i: (0, 0)),
            pl.BlockSpec((ER, D), lambda i: (0, 0)),
            pl.BlockSpec((ER, D), lambda i: (0, 0)),
        ],
        out_specs=pl.BlockSpec((TOKEN_TILE, D), lambda i: (i, 0)),
        out_shape=jax.ShapeDtypeStruct((B * N, D), x.dtype),
    )(x2, xb, Wg, wu_all, wv_all, wo_all)
    return out.reshape(B, N, D)
